# baseline (speedup 1.0000x reference)
import functools

import jax
import jax.numpy as jnp
from jax import lax
from jax.experimental import pallas as pl
from jax.experimental.pallas import tpu as pltpu
from jax.experimental.pallas import tpu_sc as plsc


def _add_body(a_ref, b_ref, o_ref):
    o_ref[...] = a_ref[...] + b_ref[...]


def _dense_add(a, b, rows):
    m, d = a.shape
    assert m % rows == 0
    return pl.pallas_call(
        _add_body,
        grid=(m // rows,),
        in_specs=[
            pl.BlockSpec((rows, d), lambda i: (i, 0)),
            pl.BlockSpec((rows, d), lambda i: (i, 0)),
        ],
        out_specs=pl.BlockSpec((rows, d), lambda i: (i, 0)),
        out_shape=jax.ShapeDtypeStruct((m, d), a.dtype),
    )(a, b)


@functools.cache
def _make_sc_scatter(b, d):
    num_cores, num_subcores, lanes = 2, 16, 16
    nw = num_cores * num_subcores
    b_per_w = b // nw
    mesh = plsc.VectorSubcoreMesh(
        core_axis_name="c", subcore_axis_name="s",
        num_cores=num_cores, num_subcores=num_subcores,
    )

    @functools.partial(
        pl.kernel,
        mesh=mesh,
        out_type=(),
        scratch_types=[
            pltpu.VMEM((b_per_w,), jnp.int32),
            pltpu.VMEM((b_per_w // 2, d), jnp.float32),
            pltpu.VMEM((b_per_w // 2, d), jnp.float32),
            pltpu.SemaphoreType.DMA,
        ],
    )
    def sc_scatter(y_hbm, c2_hbm, idx_hbm, out_ref, idx_v, y_v, c2_v, sem):
        wid = lax.axis_index("s") * num_cores + lax.axis_index("c")
        base = wid * b_per_w
        half = b_per_w // 2
        pltpu.sync_copy(idx_hbm.at[pl.ds(base, b_per_w)], idx_v)
        for h in range(2):
            r0 = base + h * half
            pltpu.sync_copy(y_hbm.at[pl.ds(r0, half)], y_v)
            pltpu.sync_copy(c2_hbm.at[pl.ds(r0, half)], c2_v)

            @pl.loop(0, half)
            def _row(i):
                for k in range(d // lanes):
                    sl = pl.ds(k * lanes, lanes)
                    y_v[i, sl] = y_v[i, sl] + c2_v[i, sl]

            @pl.loop(0, half // lanes)
            def _grp(g):
                vec = idx_v[pl.ds(h * half + g * lanes, lanes)]
                for k in range(lanes):
                    pltpu.async_copy(
                        y_v.at[pl.ds(g * lanes + k, 1)],
                        out_ref.at[pl.ds(vec[k], 1)],
                        sem,
                    )

            pltpu.make_async_copy(y_v, out_ref.at[pl.ds(0, half)], sem).wait()

    return sc_scatter


def kernel(x, y, c1, c2, index):
    dense = _dense_add(x, c1, rows=10000)
    out_ref = jax.new_ref(dense)
    _make_sc_scatter(y.shape[0], y.shape[1])(y, c2, index, out_ref)
    return out_ref[...]
